# baseline (speedup 1.0000x reference)
import functools

import jax
import jax.numpy as jnp
from jax import lax
from jax.experimental import pallas as pl
from jax.experimental.pallas import tpu as pltpu
from jax.experimental.pallas import tpu_sc as plsc

B, L, D = 4096, 200, 32
NC, NS = 2, 16
NW = NC * NS
EPW = B // NW
E = 4
NCHUNK = EPW // E
IW = 100
NIDX = E * L // IW
RPW = EPW * L // IW
RPC = E * L
INV_L = 1.0 / L

_mesh = plsc.VectorSubcoreMesh(core_axis_name="c", subcore_axis_name="s")


@functools.partial(
    pl.kernel,
    out_type=jax.ShapeDtypeStruct((B, D), jnp.float32),
    mesh=_mesh,
    compiler_params=pltpu.CompilerParams(use_tc_tiling_on_sc=False),
    scratch_types=[
        pltpu.VMEM((RPW, IW), jnp.int32),
        pltpu.VMEM((RPC, D), jnp.float32),
        pltpu.VMEM((RPC, D), jnp.float32),
        pltpu.VMEM((EPW, D), jnp.float32),
        pltpu.SemaphoreType.DMA,
        pltpu.SemaphoreType.DMA,
    ],
)
def _embed_mean(idx_hbm, table_hbm, out_hbm, idx_v, rows0, rows1, out_v,
                sem0, sem1):
    wid = lax.axis_index("s") * NC + lax.axis_index("c")
    elem0 = wid * EPW

    pltpu.sync_copy(idx_hbm.at[pl.ds(wid * RPW, RPW)], idx_v)

    def issue(c, rows, sem):
        for j in range(NIDX):
            pltpu.async_copy(
                table_hbm.at[idx_v.at[c * NIDX + j]],
                rows.at[pl.ds(j * IW, IW)],
                sem,
            )

    def drain(rows, sem):
        pltpu.make_async_copy(table_hbm.at[pl.ds(0, RPC)], rows, sem).wait()

    def reduce_store(c, rows):
        for e in range(E):
            def red(r, acc):
                a0, a1, b0, b1 = acc
                row = e * L + 2 * r
                a0 = a0 + rows[row, pl.ds(0, 16)]
                a1 = a1 + rows[row, pl.ds(16, 16)]
                b0 = b0 + rows[row + 1, pl.ds(0, 16)]
                b1 = b1 + rows[row + 1, pl.ds(16, 16)]
                return (a0, a1, b0, b1)

            z = jnp.zeros((16,), jnp.float32)
            a0, a1, b0, b1 = lax.fori_loop(0, L // 2, red, (z, z, z, z),
                                           unroll=10)
            el = c * E + e
            out_v[el, pl.ds(0, 16)] = a0 + b0
            out_v[el, pl.ds(16, 16)] = a1 + b1

    issue(0, rows0, sem0)
    issue(1, rows1, sem1)

    def pair_body(i, carry):
        c = 2 * i
        drain(rows0, sem0)
        reduce_store(c, rows0)
        issue(c + 2, rows0, sem0)
        drain(rows1, sem1)
        reduce_store(c + 1, rows1)
        issue(c + 3, rows1, sem1)
        return carry

    lax.fori_loop(0, NCHUNK // 2 - 1, pair_body, 0)

    drain(rows0, sem0)
    reduce_store(NCHUNK - 2, rows0)
    drain(rows1, sem1)
    reduce_store(NCHUNK - 1, rows1)

    pltpu.sync_copy(out_v, out_hbm.at[pl.ds(elem0, EPW)])


def kernel(input, table):
    idx2 = input.astype(jnp.int32).reshape(B * L // IW, IW)
    tbl = table * jnp.float32(INV_L)
    return _embed_mean(idx2, tbl)


# speedup vs baseline: 1.1431x; 1.1431x over previous
import functools

import jax
import jax.numpy as jnp
from jax import lax
from jax.experimental import pallas as pl
from jax.experimental.pallas import tpu as pltpu
from jax.experimental.pallas import tpu_sc as plsc

B, L, D = 4096, 200, 32
V = 1000000
NC, NS = 2, 16
NW = NC * NS
EPW = B // NW
E = 1
NCHUNK = EPW // E
IW = 100
NIDX = E * L // IW
RPW = EPW * L // IW
RPC = E * L
TW = 128
INV_L = 1.0 / L

BLK = 4096
NBLK = -(-V // BLK)

_mesh = plsc.VectorSubcoreMesh(core_axis_name="c", subcore_axis_name="s")


def _widen_body(x_ref, o_ref):
    xt = x_ref[...].T
    o_ref[...] = jnp.concatenate([xt, xt, xt, xt], axis=1)


_widen = pl.pallas_call(
    _widen_body,
    grid=(NBLK,),
    in_specs=[pl.BlockSpec((32, BLK), lambda i: (0, i))],
    out_specs=pl.BlockSpec((BLK, TW), lambda i: (i, 0)),
    out_shape=jax.ShapeDtypeStruct((V, TW), jnp.float32),
)


@functools.partial(
    pl.kernel,
    out_type=jax.ShapeDtypeStruct((B, D), jnp.float32),
    mesh=_mesh,
    compiler_params=pltpu.CompilerParams(use_tc_tiling_on_sc=True),
    scratch_types=[
        pltpu.VMEM((RPW, IW), jnp.int32),
        pltpu.VMEM((RPC, TW), jnp.float32),
        pltpu.VMEM((RPC, TW), jnp.float32),
        pltpu.VMEM((EPW, D), jnp.float32),
        pltpu.SemaphoreType.DMA,
        pltpu.SemaphoreType.DMA,
    ],
)
def _embed_mean(idx_hbm, table_hbm, out_hbm, idx_v, rows0, rows1, out_v,
                sem0, sem1):
    wid = lax.axis_index("s") * NC + lax.axis_index("c")
    elem0 = wid * EPW

    pltpu.sync_copy(idx_hbm.at[pl.ds(wid * RPW, RPW)], idx_v)

    def issue(c, rows, sem):
        for j in range(NIDX):
            pltpu.async_copy(
                table_hbm.at[idx_v.at[c * NIDX + j]],
                rows.at[pl.ds(j * IW, IW)],
                sem,
            )

    def drain(rows, sem):
        pltpu.make_async_copy(table_hbm.at[pl.ds(0, RPC)], rows, sem).wait()

    def reduce_store(c, rows):
        for e in range(E):
            def red(r, acc):
                a0, a1, b0, b1 = acc
                row = e * L + 2 * r
                a0 = a0 + rows[row, pl.ds(0, 16)]
                a1 = a1 + rows[row, pl.ds(16, 16)]
                b0 = b0 + rows[row + 1, pl.ds(0, 16)]
                b1 = b1 + rows[row + 1, pl.ds(16, 16)]
                return (a0, a1, b0, b1)

            z = jnp.zeros((16,), jnp.float32)
            a0, a1, b0, b1 = lax.fori_loop(0, L // 2, red, (z, z, z, z),
                                           unroll=10)
            el = c * E + e
            out_v[el, pl.ds(0, 16)] = (a0 + b0) * INV_L
            out_v[el, pl.ds(16, 16)] = (a1 + b1) * INV_L

    issue(0, rows0, sem0)
    issue(1, rows1, sem1)

    def pair_body(i, carry):
        c = 2 * i
        drain(rows0, sem0)
        reduce_store(c, rows0)
        issue(c + 2, rows0, sem0)
        drain(rows1, sem1)
        reduce_store(c + 1, rows1)
        issue(c + 3, rows1, sem1)
        return carry

    lax.fori_loop(0, NCHUNK // 2 - 1, pair_body, 0)

    drain(rows0, sem0)
    reduce_store(NCHUNK - 2, rows0)
    drain(rows1, sem1)
    reduce_store(NCHUNK - 1, rows1)

    pltpu.sync_copy(out_v, out_hbm.at[pl.ds(elem0, EPW)])


def kernel(input, table):
    idx2 = input.astype(jnp.int32).reshape(B * L // IW, IW)
    wide = _widen(table.T)
    return _embed_mean(idx2, wide)


# speedup vs baseline: 1.6659x; 1.4574x over previous
import functools

import jax
import jax.numpy as jnp
from jax import lax
from jax.experimental import pallas as pl
from jax.experimental.pallas import tpu as pltpu
from jax.experimental.pallas import tpu_sc as plsc

B, L, D = 4096, 200, 32
V = 1000000
NC, NS = 2, 16
NW = NC * NS
EPW = B // NW
E = 1
NCHUNK = EPW // E
IW = 100
NIDX = E * L // IW
RPW = EPW * L // IW
RPC = E * L
TW = 128
INV_L = 1.0 / L

BLK = 4096
NBLK = -(-V // BLK)

_mesh = plsc.VectorSubcoreMesh(core_axis_name="c", subcore_axis_name="s")


def _widen_body(x_ref, o_ref):
    x = x_ref[...]
    qf = lax.broadcasted_iota(jnp.int32, (32, TW), 1) % 32
    ff = lax.broadcasted_iota(jnp.int32, (32, TW), 0)
    rep = (qf == ff).astype(jnp.float32)
    o_ref[...] = lax.dot_general(x, rep, (((0,), (0,)), ((), ())),
                                 preferred_element_type=jnp.float32)


_widen = pl.pallas_call(
    _widen_body,
    grid=(NBLK,),
    in_specs=[pl.BlockSpec((32, BLK), lambda i: (0, i))],
    out_specs=pl.BlockSpec((BLK, TW), lambda i: (i, 0)),
    out_shape=jax.ShapeDtypeStruct((V, TW), jnp.float32),
)


@functools.partial(
    pl.kernel,
    out_type=jax.ShapeDtypeStruct((B, D), jnp.float32),
    mesh=_mesh,
    compiler_params=pltpu.CompilerParams(use_tc_tiling_on_sc=True),
    scratch_types=[
        pltpu.VMEM((RPW, IW), jnp.int32),
        pltpu.VMEM((RPC, TW), jnp.float32),
        pltpu.VMEM((RPC, TW), jnp.float32),
        pltpu.VMEM((EPW, D), jnp.float32),
        pltpu.SemaphoreType.DMA,
        pltpu.SemaphoreType.DMA,
    ],
)
def _embed_mean(idx_hbm, table_hbm, out_hbm, idx_v, rows0, rows1, out_v,
                sem0, sem1):
    wid = lax.axis_index("s") * NC + lax.axis_index("c")
    elem0 = wid * EPW

    pltpu.sync_copy(idx_hbm.at[pl.ds(wid * RPW, RPW)], idx_v)

    def issue(c, rows, sem):
        for j in range(NIDX):
            pltpu.async_copy(
                table_hbm.at[idx_v.at[c * NIDX + j]],
                rows.at[pl.ds(j * IW, IW)],
                sem,
            )

    def drain(rows, sem):
        pltpu.make_async_copy(table_hbm.at[pl.ds(0, RPC)], rows, sem).wait()

    def reduce_store(c, rows):
        for e in range(E):
            def red(r, acc):
                a0, a1, b0, b1 = acc
                row = e * L + 2 * r
                a0 = a0 + rows[row, pl.ds(0, 16)]
                a1 = a1 + rows[row, pl.ds(16, 16)]
                b0 = b0 + rows[row + 1, pl.ds(0, 16)]
                b1 = b1 + rows[row + 1, pl.ds(16, 16)]
                return (a0, a1, b0, b1)

            z = jnp.zeros((16,), jnp.float32)
            a0, a1, b0, b1 = lax.fori_loop(0, L // 2, red, (z, z, z, z),
                                           unroll=10)
            el = c * E + e
            out_v[el, pl.ds(0, 16)] = (a0 + b0) * INV_L
            out_v[el, pl.ds(16, 16)] = (a1 + b1) * INV_L

    issue(0, rows0, sem0)
    issue(1, rows1, sem1)

    def pair_body(i, carry):
        c = 2 * i
        drain(rows0, sem0)
        reduce_store(c, rows0)
        issue(c + 2, rows0, sem0)
        drain(rows1, sem1)
        reduce_store(c + 1, rows1)
        issue(c + 3, rows1, sem1)
        return carry

    lax.fori_loop(0, NCHUNK // 2 - 1, pair_body, 0)

    drain(rows0, sem0)
    reduce_store(NCHUNK - 2, rows0)
    drain(rows1, sem1)
    reduce_store(NCHUNK - 1, rows1)

    pltpu.sync_copy(out_v, out_hbm.at[pl.ds(elem0, EPW)])


def kernel(input, table):
    idx2 = input.astype(jnp.int32).reshape(B * L // IW, IW)
    wide = _widen(table.T)
    return _embed_mean(idx2, wide)


# speedup vs baseline: 1.9301x; 1.1586x over previous
import functools

import jax
import jax.numpy as jnp
from jax import lax
from jax.experimental import pallas as pl
from jax.experimental.pallas import tpu as pltpu
from jax.experimental.pallas import tpu_sc as plsc

B, L, D = 4096, 200, 32
V = 1000000
NC, NS = 2, 16
NW = NC * NS
EPW = B // NW
E = 1
NCHUNK = EPW // E
IW = 100
NIDX = E * L // IW
RPW = EPW * L // IW
RPC = E * L
TW = 128
INV_L = 1.0 / L

BLK = 8192
NBLK = -(-V // BLK)

_mesh = plsc.VectorSubcoreMesh(core_axis_name="c", subcore_axis_name="s")


def _widen_body(x_ref, o_ref):
    x = x_ref[...]
    qf = lax.broadcasted_iota(jnp.int32, (32, TW), 1) % 32
    ff = lax.broadcasted_iota(jnp.int32, (32, TW), 0)
    rep = (qf == ff).astype(jnp.float32)
    o_ref[...] = lax.dot_general(x, rep, (((0,), (0,)), ((), ())),
                                 preferred_element_type=jnp.float32)


_widen = pl.pallas_call(
    _widen_body,
    grid=(NBLK,),
    in_specs=[pl.BlockSpec((32, BLK), lambda i: (0, i))],
    out_specs=pl.BlockSpec((BLK, TW), lambda i: (i, 0)),
    out_shape=jax.ShapeDtypeStruct((V, TW), jnp.float32),
)


@functools.partial(
    pl.kernel,
    out_type=jax.ShapeDtypeStruct((B, D), jnp.float32),
    mesh=_mesh,
    compiler_params=pltpu.CompilerParams(use_tc_tiling_on_sc=True),
    scratch_types=[
        pltpu.VMEM((RPW, IW), jnp.int32),
        pltpu.VMEM((RPC, TW), jnp.float32),
        pltpu.VMEM((RPC, TW), jnp.float32),
        pltpu.VMEM((EPW, D), jnp.float32),
        pltpu.SemaphoreType.DMA,
        pltpu.SemaphoreType.DMA,
    ],
)
def _embed_mean(idx_hbm, table_hbm, out_hbm, idx_v, rows0, rows1, out_v,
                sem0, sem1):
    wid = lax.axis_index("s") * NC + lax.axis_index("c")
    elem0 = wid * EPW

    pltpu.sync_copy(idx_hbm.at[pl.ds(wid * RPW, RPW)], idx_v)

    def issue(c, rows, sem):
        for j in range(NIDX):
            pltpu.async_copy(
                table_hbm.at[idx_v.at[c * NIDX + j]],
                rows.at[pl.ds(j * IW, IW)],
                sem,
            )

    def drain(rows, sem):
        pltpu.make_async_copy(table_hbm.at[pl.ds(0, RPC)], rows, sem).wait()

    def reduce_store(c, rows):
        for e in range(E):
            def red(r, acc):
                a0, a1, b0, b1 = acc
                row = e * L + 2 * r
                a0 = a0 + rows[row, pl.ds(0, 16)]
                a1 = a1 + rows[row, pl.ds(16, 16)]
                b0 = b0 + rows[row + 1, pl.ds(0, 16)]
                b1 = b1 + rows[row + 1, pl.ds(16, 16)]
                return (a0, a1, b0, b1)

            z = jnp.zeros((16,), jnp.float32)
            a0, a1, b0, b1 = lax.fori_loop(0, L // 2, red, (z, z, z, z),
                                           unroll=10)
            el = c * E + e
            out_v[el, pl.ds(0, 16)] = (a0 + b0) * INV_L
            out_v[el, pl.ds(16, 16)] = (a1 + b1) * INV_L

    issue(0, rows0, sem0)
    issue(1, rows1, sem1)

    def pair_body(i, carry):
        c = 2 * i
        drain(rows0, sem0)
        reduce_store(c, rows0)
        issue(c + 2, rows0, sem0)
        drain(rows1, sem1)
        reduce_store(c + 1, rows1)
        issue(c + 3, rows1, sem1)
        return carry

    lax.fori_loop(0, NCHUNK // 2 - 1, pair_body, 0)

    drain(rows0, sem0)
    reduce_store(NCHUNK - 2, rows0)
    drain(rows1, sem1)
    reduce_store(NCHUNK - 1, rows1)

    pltpu.sync_copy(out_v, out_hbm.at[pl.ds(elem0, EPW)])


def kernel(input, table):
    idx2 = input.astype(jnp.int32).reshape(B * L // IW, IW)
    wide = _widen(table.T)
    return _embed_mean(idx2, wide)


# speedup vs baseline: 2.1047x; 1.0905x over previous
import functools

import jax
import jax.numpy as jnp
from jax import lax
from jax.experimental import pallas as pl
from jax.experimental.pallas import tpu as pltpu
from jax.experimental.pallas import tpu_sc as plsc

B, L, D = 4096, 200, 32
V = 1000000
NC, NS = 2, 16
NW = NC * NS
EPW = B // NW
E = 1
NCHUNK = EPW // E
IW = 100
NIDX = E * L // IW
RPW = EPW * L // IW
RPC = E * L
TW = 128
INV_L = 1.0 / L

BLK = 16384
NBLK = -(-V // BLK)

_mesh = plsc.VectorSubcoreMesh(core_axis_name="c", subcore_axis_name="s")


def _widen_body(x_ref, o_ref):
    x = x_ref[...]
    qf = lax.broadcasted_iota(jnp.int32, (32, TW), 1) % 32
    ff = lax.broadcasted_iota(jnp.int32, (32, TW), 0)
    rep = (qf == ff).astype(jnp.float32)
    o_ref[...] = lax.dot_general(x, rep, (((0,), (0,)), ((), ())),
                                 preferred_element_type=jnp.float32)


_widen = pl.pallas_call(
    _widen_body,
    grid=(NBLK,),
    in_specs=[pl.BlockSpec((32, BLK), lambda i: (0, i))],
    out_specs=pl.BlockSpec((BLK, TW), lambda i: (i, 0)),
    out_shape=jax.ShapeDtypeStruct((V, TW), jnp.float32),
)


@functools.partial(
    pl.kernel,
    out_type=jax.ShapeDtypeStruct((B, D), jnp.float32),
    mesh=_mesh,
    compiler_params=pltpu.CompilerParams(use_tc_tiling_on_sc=True),
    scratch_types=[
        pltpu.VMEM((RPW, IW), jnp.int32),
        pltpu.VMEM((RPC, TW), jnp.float32),
        pltpu.VMEM((RPC, TW), jnp.float32),
        pltpu.VMEM((EPW, D), jnp.float32),
        pltpu.SemaphoreType.DMA,
        pltpu.SemaphoreType.DMA,
    ],
)
def _embed_mean(idx_hbm, table_hbm, out_hbm, idx_v, rows0, rows1, out_v,
                sem0, sem1):
    wid = lax.axis_index("s") * NC + lax.axis_index("c")
    elem0 = wid * EPW

    pltpu.sync_copy(idx_hbm.at[pl.ds(wid * RPW, RPW)], idx_v)

    def issue(c, rows, sem):
        for j in range(NIDX):
            pltpu.async_copy(
                table_hbm.at[idx_v.at[c * NIDX + j]],
                rows.at[pl.ds(j * IW, IW)],
                sem,
            )

    def drain(rows, sem):
        pltpu.make_async_copy(table_hbm.at[pl.ds(0, RPC)], rows, sem).wait()

    def reduce_store(c, rows):
        for e in range(E):
            def red(r, acc):
                a0, a1, b0, b1 = acc
                row = e * L + 2 * r
                a0 = a0 + rows[row, pl.ds(0, 16)]
                a1 = a1 + rows[row, pl.ds(16, 16)]
                b0 = b0 + rows[row + 1, pl.ds(0, 16)]
                b1 = b1 + rows[row + 1, pl.ds(16, 16)]
                return (a0, a1, b0, b1)

            z = jnp.zeros((16,), jnp.float32)
            a0, a1, b0, b1 = lax.fori_loop(0, L // 2, red, (z, z, z, z),
                                           unroll=10)
            el = c * E + e
            out_v[el, pl.ds(0, 16)] = (a0 + b0) * INV_L
            out_v[el, pl.ds(16, 16)] = (a1 + b1) * INV_L

    issue(0, rows0, sem0)
    issue(1, rows1, sem1)

    def pair_body(i, carry):
        c = 2 * i
        drain(rows0, sem0)
        reduce_store(c, rows0)
        issue(c + 2, rows0, sem0)
        drain(rows1, sem1)
        reduce_store(c + 1, rows1)
        issue(c + 3, rows1, sem1)
        return carry

    lax.fori_loop(0, NCHUNK // 2 - 1, pair_body, 0)

    drain(rows0, sem0)
    reduce_store(NCHUNK - 2, rows0)
    drain(rows1, sem1)
    reduce_store(NCHUNK - 1, rows1)

    pltpu.sync_copy(out_v, out_hbm.at[pl.ds(elem0, EPW)])


def kernel(input, table):
    idx2 = input.astype(jnp.int32).reshape(B * L // IW, IW)
    wide = _widen(table.T)
    return _embed_mean(idx2, wide)


# speedup vs baseline: 2.1336x; 1.0137x over previous
import functools

import jax
import jax.numpy as jnp
from jax import lax
from jax.experimental import pallas as pl
from jax.experimental.pallas import tpu as pltpu
from jax.experimental.pallas import tpu_sc as plsc

B, L, D = 4096, 200, 32
V = 1000000
NC, NS = 2, 16
NW = NC * NS
EPW = B // NW
E = 1
NCHUNK = EPW // E
IW = 100
NIDX = E * L // IW
RPW = EPW * L // IW
RPC = E * L
TW = 128
INV_L = 1.0 / L

BLK = 32768
NBLK = -(-V // BLK)

_mesh = plsc.VectorSubcoreMesh(core_axis_name="c", subcore_axis_name="s")


def _widen_body(x_ref, o_ref):
    x = x_ref[...]
    qf = lax.broadcasted_iota(jnp.int32, (32, TW), 1) % 32
    ff = lax.broadcasted_iota(jnp.int32, (32, TW), 0)
    rep = (qf == ff).astype(jnp.float32)
    o_ref[...] = lax.dot_general(x, rep, (((0,), (0,)), ((), ())),
                                 preferred_element_type=jnp.float32)


_widen = pl.pallas_call(
    _widen_body,
    grid=(NBLK,),
    in_specs=[pl.BlockSpec((32, BLK), lambda i: (0, i))],
    out_specs=pl.BlockSpec((BLK, TW), lambda i: (i, 0)),
    out_shape=jax.ShapeDtypeStruct((V, TW), jnp.float32),
)


@functools.partial(
    pl.kernel,
    out_type=jax.ShapeDtypeStruct((B, D), jnp.float32),
    mesh=_mesh,
    compiler_params=pltpu.CompilerParams(use_tc_tiling_on_sc=True),
    scratch_types=[
        pltpu.VMEM((RPW, IW), jnp.int32),
        pltpu.VMEM((RPC, TW), jnp.float32),
        pltpu.VMEM((RPC, TW), jnp.float32),
        pltpu.VMEM((EPW, D), jnp.float32),
        pltpu.SemaphoreType.DMA,
        pltpu.SemaphoreType.DMA,
    ],
)
def _embed_mean(idx_hbm, table_hbm, out_hbm, idx_v, rows0, rows1, out_v,
                sem0, sem1):
    wid = lax.axis_index("s") * NC + lax.axis_index("c")
    elem0 = wid * EPW

    pltpu.sync_copy(idx_hbm.at[pl.ds(wid * RPW, RPW)], idx_v)

    def issue(c, rows, sem):
        for j in range(NIDX):
            pltpu.async_copy(
                table_hbm.at[idx_v.at[c * NIDX + j]],
                rows.at[pl.ds(j * IW, IW)],
                sem,
            )

    def drain(rows, sem):
        pltpu.make_async_copy(table_hbm.at[pl.ds(0, RPC)], rows, sem).wait()

    def reduce_store(c, rows):
        for e in range(E):
            def red(r, acc):
                a0, a1, b0, b1 = acc
                row = e * L + 2 * r
                a0 = a0 + rows[row, pl.ds(0, 16)]
                a1 = a1 + rows[row, pl.ds(16, 16)]
                b0 = b0 + rows[row + 1, pl.ds(0, 16)]
                b1 = b1 + rows[row + 1, pl.ds(16, 16)]
                return (a0, a1, b0, b1)

            z = jnp.zeros((16,), jnp.float32)
            a0, a1, b0, b1 = lax.fori_loop(0, L // 2, red, (z, z, z, z),
                                           unroll=10)
            el = c * E + e
            out_v[el, pl.ds(0, 16)] = (a0 + b0) * INV_L
            out_v[el, pl.ds(16, 16)] = (a1 + b1) * INV_L

    issue(0, rows0, sem0)
    issue(1, rows1, sem1)

    def pair_body(i, carry):
        c = 2 * i
        drain(rows0, sem0)
        reduce_store(c, rows0)
        issue(c + 2, rows0, sem0)
        drain(rows1, sem1)
        reduce_store(c + 1, rows1)
        issue(c + 3, rows1, sem1)
        return carry

    lax.fori_loop(0, NCHUNK // 2 - 1, pair_body, 0)

    drain(rows0, sem0)
    reduce_store(NCHUNK - 2, rows0)
    drain(rows1, sem1)
    reduce_store(NCHUNK - 1, rows1)

    pltpu.sync_copy(out_v, out_hbm.at[pl.ds(elem0, EPW)])


def kernel(input, table):
    idx2 = input.astype(jnp.int32).reshape(B * L // IW, IW)
    wide = _widen(table.T)
    return _embed_mean(idx2, wide)
